# Initial kernel scaffold; baseline (speedup 1.0000x reference)
import functools

import jax
import jax.numpy as jnp
from jax import lax
from jax.experimental import pallas as pl
from jax.experimental.pallas import tpu as pltpu
from jax.experimental.pallas import tpu_sc as plsc

NC = 2
NS = 16
NW = NC * NS

D = 64
B = 16384 * 50
B_PER_W = B // NW
CHUNK = 512
N_CHUNKS = B_PER_W // CHUNK

_mesh = plsc.VectorSubcoreMesh(core_axis_name="c", subcore_axis_name="s")


@functools.partial(
    pl.kernel,
    out_type=jax.ShapeDtypeStruct((B, D), jnp.float32),
    mesh=_mesh,
    scratch_types=[
        pltpu.VMEM((CHUNK,), jnp.int32),
        pltpu.VMEM((CHUNK, D), jnp.float32),
        pltpu.SemaphoreType.DMA,
    ],
)
def _gather_kernel(idx_hbm, table_hbm, out_hbm, idx_v, rows_v, sem):
    wid = lax.axis_index("s") * NC + lax.axis_index("c")
    wbase = wid * B_PER_W

    def body(i, carry):
        base = pl.multiple_of(wbase + i * CHUNK, CHUNK)
        pltpu.sync_copy(idx_hbm.at[pl.ds(base, CHUNK)], idx_v)
        pltpu.async_copy(table_hbm.at[idx_v], rows_v, sem).wait()
        pltpu.sync_copy(rows_v, out_hbm.at[pl.ds(base, CHUNK)])
        return carry

    lax.fori_loop(0, N_CHUNKS, body, 0)


def kernel(token_ids, weight):
    flat = token_ids.reshape(-1).astype(jnp.int32)
    out = _gather_kernel(flat, weight)
    return out.reshape(token_ids.shape + (weight.shape[1],))


# speedup vs baseline: 1.7954x; 1.7954x over previous
import functools

import jax
import jax.numpy as jnp
from jax import lax
from jax.experimental import pallas as pl
from jax.experimental.pallas import tpu as pltpu
from jax.experimental.pallas import tpu_sc as plsc

NC = 2
NS = 16
NW = NC * NS

D = 64
B = 16384 * 50
B_PER_W = B // NW
CHUNK = 512
N_CHUNKS = B_PER_W // CHUNK

_mesh = plsc.VectorSubcoreMesh(core_axis_name="c", subcore_axis_name="s")


@functools.partial(
    pl.kernel,
    out_type=jax.ShapeDtypeStruct((B, D), jnp.float32),
    mesh=_mesh,
    scratch_types=[
        pltpu.VMEM((CHUNK,), jnp.int32),
        pltpu.VMEM((CHUNK, D), jnp.float32),
        pltpu.SemaphoreType.DMA,
    ],
    compiler_params=pltpu.CompilerParams(use_tc_tiling_on_sc=False),
)
def _gather_kernel(idx_hbm, table_hbm, out_hbm, idx_v, rows_v, sem):
    wid = lax.axis_index("s") * NC + lax.axis_index("c")
    wbase = wid * B_PER_W

    def body(i, carry):
        base = pl.multiple_of(wbase + i * CHUNK, CHUNK)
        pltpu.sync_copy(idx_hbm.at[pl.ds(base, CHUNK)], idx_v)
        pltpu.async_copy(table_hbm.at[idx_v], rows_v, sem).wait()
        pltpu.sync_copy(rows_v, out_hbm.at[pl.ds(base, CHUNK)])
        return carry

    lax.fori_loop(0, N_CHUNKS, body, 0)


def kernel(token_ids, weight):
    flat = token_ids.reshape(-1).astype(jnp.int32)
    out = _gather_kernel(flat, weight)
    return out.reshape(token_ids.shape + (weight.shape[1],))


# speedup vs baseline: 1.8772x; 1.0455x over previous
import functools

import jax
import jax.numpy as jnp
from jax import lax
from jax.experimental import pallas as pl
from jax.experimental.pallas import tpu as pltpu
from jax.experimental.pallas import tpu_sc as plsc

NC = 2
NS = 16
NW = NC * NS

D = 64
B = 16384 * 50
B_PER_W = B // NW
CHUNK = 512
N_CHUNKS = B_PER_W // CHUNK
NBUF = 2
MAIN_G = (N_CHUNKS - NBUF) // NBUF
assert N_CHUNKS % NBUF == 0 and B_PER_W % CHUNK == 0

_mesh = plsc.VectorSubcoreMesh(core_axis_name="c", subcore_axis_name="s")


@functools.partial(
    pl.kernel,
    out_type=jax.ShapeDtypeStruct((B, D), jnp.float32),
    mesh=_mesh,
    scratch_types=(
        [pltpu.VMEM((CHUNK,), jnp.int32) for _ in range(NBUF)]
        + [pltpu.VMEM((CHUNK, D), jnp.float32) for _ in range(NBUF)]
        + [pltpu.SemaphoreType.DMA for _ in range(2 * NBUF)]
    ),
    compiler_params=pltpu.CompilerParams(use_tc_tiling_on_sc=False),
)
def _gather_kernel(idx_hbm, table_hbm, out_hbm, *scratch):
    idx_bufs = scratch[:NBUF]
    row_bufs = scratch[NBUF:2 * NBUF]
    gsems = scratch[2 * NBUF:3 * NBUF]
    osems = scratch[3 * NBUF:]

    wid = lax.axis_index("s") * NC + lax.axis_index("c")
    wbase = wid * B_PER_W

    def chunk_slice(i):
        return pl.ds(pl.multiple_of(wbase + i * CHUNK, CHUNK), CHUNK)

    for b in range(NBUF):
        pltpu.sync_copy(idx_hbm.at[chunk_slice(b)], idx_bufs[b])
        pltpu.async_copy(table_hbm.at[idx_bufs[b]], row_bufs[b], gsems[b])

    @pl.loop(0, MAIN_G)
    def main(g):
        for b in range(NBUF):
            i = g * NBUF + b
            pltpu.make_async_copy(
                table_hbm.at[idx_bufs[b]], row_bufs[b], gsems[b]).wait()
            pltpu.async_copy(row_bufs[b], out_hbm.at[chunk_slice(i)], osems[b])
            pltpu.sync_copy(idx_hbm.at[chunk_slice(i + NBUF)], idx_bufs[b])
            pltpu.make_async_copy(
                row_bufs[b], out_hbm.at[chunk_slice(i)], osems[b]).wait()
            pltpu.async_copy(table_hbm.at[idx_bufs[b]], row_bufs[b], gsems[b])

    for b in range(NBUF):
        i = MAIN_G * NBUF + b
        pltpu.make_async_copy(
            table_hbm.at[idx_bufs[b]], row_bufs[b], gsems[b]).wait()
        pltpu.async_copy(row_bufs[b], out_hbm.at[chunk_slice(i)], osems[b])
    for b in range(NBUF):
        i = MAIN_G * NBUF + b
        pltpu.make_async_copy(
            row_bufs[b], out_hbm.at[chunk_slice(i)], osems[b]).wait()


def kernel(token_ids, weight):
    flat = token_ids.reshape(-1).astype(jnp.int32)
    out = _gather_kernel(flat, weight)
    return out.reshape(token_ids.shape + (weight.shape[1],))
